# Initial kernel scaffold; baseline (speedup 1.0000x reference)
import jax
import jax.numpy as jnp
from jax.experimental import pallas as pl


def kernel(boxes, images, patch, scale):
    raise NotImplementedError("write your pallas kernel here")


# speedup vs baseline: 78.1273x; 78.1273x over previous
import numpy as np
import jax
import jax.numpy as jnp
from jax import lax
from jax.experimental import pallas as pl
from jax.experimental.pallas import tpu as pltpu

MIN_PATCH_AREA = 60.0
SMIN, SMAX = 20, 72
NSIZES = SMAX - SMIN + 1
TS = 72
TW = TS * 3
WR = 128
WC = 384


def _resize_mats(insz: int) -> np.ndarray:
    out = np.zeros((NSIZES, TS, insz), np.float32)
    for i, s in enumerate(range(SMIN, SMAX + 1)):
        scale = s / insz
        kernel_scale = min(scale, 1.0)
        sample_f = (np.arange(s) + 0.5) / scale - 0.5
        x = np.abs(sample_f[None, :] - np.arange(insz)[:, None]) * kernel_scale
        w = np.maximum(0.0, 1.0 - x)
        total = w.sum(axis=0, keepdims=True)
        w = np.where(np.abs(total) > 1000.0 * np.finfo(np.float32).eps,
                     w / np.where(total == 0, 1, total), 0.0)
        ok = (sample_f >= -0.5) & (sample_f <= insz - 0.5)
        w = np.where(ok[None, :], w, 0.0)
        out[i, :s, :] = w.T.astype(np.float32)
    return out


def _create_box(key, img_h, img_w, box, scale):
    ymin, xmin, ymax, xmax = box[0], box[1], box[2], box[3]
    h = ymax - ymin
    w = xmax - xmin
    area = h * w
    tol = 0.2
    patch_size = jnp.floor(jnp.sqrt(area * scale))
    k1, k2 = jax.random.split(key)
    orig_y = ymin + h / 2.0 + jax.random.uniform(
        k1, (), minval=-tol * h / 2.0, maxval=tol * h / 2.0)
    orig_x = xmin + w / 2.0 + jax.random.uniform(
        k2, (), minval=-tol * w / 2.0, maxval=tol * w / 2.0)
    ymin_p = jnp.maximum(orig_y - patch_size / 2.0, 0.0)
    xmin_p = jnp.maximum(orig_x - patch_size / 2.0, 0.0)
    ymin_p = jnp.where(ymin_p + patch_size > img_h, img_h - patch_size, ymin_p)
    xmin_p = jnp.where(xmin_p + patch_size > img_w, img_w - patch_size, xmin_p)
    return jnp.stack([ymin_p, xmin_p, patch_size, patch_size])


def _rotl(x, d):
    return lax.shift_left(x, jnp.uint32(d)) | lax.shift_right_logical(
        x, jnp.uint32(32 - d))


def _threefry_bits(k0, k1, v):
    ks0 = k0
    ks1 = k1
    ks2 = jnp.uint32(0x1BD11BDA) ^ ks0 ^ ks1
    ks = (ks0, ks1, ks2)
    rots = ((13, 15, 26, 6), (17, 29, 16, 24))
    x0 = jnp.full_like(v, ks0)
    x1 = v + ks1
    for i in range(5):
        for r in rots[i % 2]:
            x0 = x0 + x1
            x1 = _rotl(x1, r) ^ x0
        x0 = x0 + ks[(i + 1) % 3]
        x1 = x1 + ks[(i + 2) % 3] + jnp.uint32(i + 1)
    return x0 ^ x1


def _patch_body(geo_ref, fs_ref, wb_ref, img_ref, rmat_ref, patch_ref,
                out_ref):
    b = pl.program_id(0)
    img = img_ref[0]
    mean_img = jnp.mean(img)

    p0 = jnp.clip(wb_ref[b, 0] * patch_ref[0] + wb_ref[b, 3], -1.0, 1.0)
    p1 = jnp.clip(wb_ref[b, 1] * patch_ref[1] + wb_ref[b, 4], -1.0, 1.0)
    p2 = jnp.clip(wb_ref[b, 2] * patch_ref[2] + wb_ref[b, 5], -1.0, 1.0)
    mean_p = (jnp.sum(p0) + jnp.sum(p1) + jnp.sum(p2)) / (3.0 * p0.size)
    delta = mean_img - mean_p

    out_ref[0] = img

    nb = geo_ref.shape[1]
    for ni in range(nb):
        wy = geo_ref[b, ni, 0]
        wx = geo_ref[b, ni, 1]
        oy = geo_ref[b, ni, 2]
        ox = geo_ref[b, ni, 3]
        s = geo_ref[b, ni, 4]
        sidx = geo_ref[b, ni, 5]
        live = geo_ref[b, ni, 6]
        k0 = geo_ref[b, ni, 7]
        k1 = geo_ref[b, ni, 8]
        shift = fs_ref[b, ni]

        @pl.when(live == 1)
        def _():
            rmat = rmat_ref[sidx]
            rt = rmat.T
            t0 = jnp.dot(jnp.dot(rmat, p0, preferred_element_type=jnp.float32),
                         rt, preferred_element_type=jnp.float32)
            t1 = jnp.dot(jnp.dot(rmat, p1, preferred_element_type=jnp.float32),
                         rt, preferred_element_type=jnp.float32)
            t2 = jnp.dot(jnp.dot(rmat, p2, preferred_element_type=jnp.float32),
                         rt, preferred_element_type=jnp.float32)
            tile = jnp.stack([t0, t1, t2], axis=-1).reshape(TS, TW)

            ri = lax.broadcasted_iota(jnp.uint32, (TS, TW), 0)
            ci = lax.broadcasted_iota(jnp.uint32, (TS, TW), 1)
            v = jnp.uint32(3) * jnp.uint32(s) * ri + ci
            bits = _threefry_bits(jnp.uint32(k0), jnp.uint32(k1), v)
            f = lax.bitcast_convert_type(
                lax.shift_right_logical(bits, jnp.uint32(9))
                | jnp.uint32(0x3F800000), jnp.float32) - 1.0
            noise = jnp.maximum(jnp.float32(-0.01), f * 0.02 - 0.01)

            tile = jnp.clip(tile + (delta + shift) + noise, -1.0, 1.0)

            canvas = jnp.pad(tile, ((0, WR - TS), (0, WC - TW)))
            rolled = pltpu.roll(pltpu.roll(canvas, oy, 0), ox, 1)
            rows = lax.broadcasted_iota(jnp.int32, (WR, WC), 0)
            cols = lax.broadcasted_iota(jnp.int32, (WR, WC), 1)
            valid = ((rows >= oy) & (rows < oy + s)
                     & (cols >= ox) & (cols < ox + 3 * s))
            wya = pl.multiple_of(wy, 8)
            wxa = pl.multiple_of(wx, 128)
            win = out_ref[0, pl.ds(wya, WR), pl.ds(wxa, WC)]
            out_ref[0, pl.ds(wya, WR), pl.ds(wxa, WC)] = jnp.where(
                valid, rolled, win)


def kernel(boxes, images, patch, scale):
    B, H, W, C = images.shape
    NB = boxes.shape[1]
    PS = patch.shape[0]
    WF = W * C

    base = jax.random.key(42)
    kb = jax.vmap(lambda i: jax.random.fold_in(base, i))(jnp.arange(B))
    ks = jax.vmap(lambda k: jax.random.split(k, 4))(kb)
    kw, kbias, kboxes, knoise = ks[:, 0], ks[:, 1], ks[:, 2], ks[:, 3]

    w = jax.vmap(lambda k: jax.random.normal(k, (3,)))(kw) * 0.01 + 0.8
    bvec = jax.vmap(lambda k: jax.random.normal(k, (3,)))(kbias) * 0.01 - 0.2
    wb = jnp.concatenate([w, bvec], axis=1).astype(jnp.float32)

    def per_box(kboxes_b, knoise_b, boxes_b, ni):
        pb = _create_box(jax.random.fold_in(kboxes_b, ni),
                         float(H), float(W), boxes_b[ni], scale)
        ps, pw_f = pb[2], pb[3]
        ph = jnp.floor(ps).astype(jnp.int32)
        y0 = jnp.floor(pb[0]).astype(jnp.int32)
        x0 = jnp.floor(pb[1]).astype(jnp.int32)
        sidx = jnp.clip(ph - SMIN, 0, NSIZES - 1)
        s = sidx + SMIN
        live = jnp.logical_not(
            (ps * pw_f <= MIN_PATCH_AREA) | (ph <= 0)).astype(jnp.int32)
        kk1, kk2 = jax.random.split(jax.random.fold_in(knoise_b, ni))
        shift = jax.random.uniform(kk2, (), minval=-0.3, maxval=0.3)
        kd = jax.random.key_data(kk1).astype(jnp.uint32)
        wy = jnp.clip((y0 // 8) * 8, 0, H - WR)
        wx = jnp.clip(((3 * x0) // 128) * 128, 0, WF - WC)
        oy = jnp.clip(y0 - wy, 0, WR - 1)
        ox = jnp.clip(3 * x0 - wx, 0, WC - 3)
        geo = jnp.stack([wy, wx, oy, ox, s, sidx, live,
                         kd[0].astype(jnp.int32), kd[1].astype(jnp.int32)])
        return geo, shift

    geo, shift = jax.vmap(
        lambda kb_, kn_, bx_: jax.vmap(
            lambda ni: per_box(kb_, kn_, bx_, ni))(jnp.arange(NB))
    )(kboxes, knoise, boxes)
    geo = geo.astype(jnp.int32)
    shift = shift.astype(jnp.float32)

    rmats = jnp.asarray(_resize_mats(PS))
    patch_pl = patch.transpose(2, 0, 1)
    img_flat = images.reshape(B, H, WF)

    out = pl.pallas_call(
        _patch_body,
        grid=(B,),
        in_specs=[
            pl.BlockSpec(memory_space=pltpu.SMEM),
            pl.BlockSpec(memory_space=pltpu.SMEM),
            pl.BlockSpec(memory_space=pltpu.SMEM),
            pl.BlockSpec((1, H, WF), lambda b: (b, 0, 0)),
            pl.BlockSpec((NSIZES, TS, PS), lambda b: (0, 0, 0)),
            pl.BlockSpec((C, PS, PS), lambda b: (0, 0, 0)),
        ],
        out_specs=pl.BlockSpec((1, H, WF), lambda b: (b, 0, 0)),
        out_shape=jax.ShapeDtypeStruct((B, H, WF), jnp.float32),
    )(geo, shift, wb, img_flat, rmats, patch_pl)

    return out.reshape(B, H, W, C)


# speedup vs baseline: 165.4243x; 2.1174x over previous
import numpy as np
import jax
import jax.numpy as jnp
from jax import lax
from jax.experimental import pallas as pl
from jax.experimental.pallas import tpu as pltpu

MIN_PATCH_AREA = 60.0
SMIN, SMAX = 20, 72
NSIZES = SMAX - SMIN + 1
TS = 72
TW = TS * 3
WR = 128
WC = 384


def _resize_mats(insz: int) -> np.ndarray:
    out = np.zeros((NSIZES, TS, insz), np.float32)
    for i, s in enumerate(range(SMIN, SMAX + 1)):
        scale = s / insz
        kernel_scale = min(scale, 1.0)
        sample_f = (np.arange(s) + 0.5) / scale - 0.5
        x = np.abs(sample_f[None, :] - np.arange(insz)[:, None]) * kernel_scale
        w = np.maximum(0.0, 1.0 - x)
        total = w.sum(axis=0, keepdims=True)
        w = np.where(np.abs(total) > 1000.0 * np.finfo(np.float32).eps,
                     w / np.where(total == 0, 1, total), 0.0)
        ok = (sample_f >= -0.5) & (sample_f <= insz - 0.5)
        w = np.where(ok[None, :], w, 0.0)
        out[i, :s, :] = w.T.astype(np.float32)
    return out


def _interleave_mats(mats: np.ndarray) -> np.ndarray:
    insz = mats.shape[2]
    out = np.zeros((3, NSIZES, insz, TW), np.float32)
    for si in range(NSIZES):
        at = mats[si].T
        for c in range(3):
            out[c, si, :, c::3] = at
    return out


def _create_box(key, img_h, img_w, box, scale):
    ymin, xmin, ymax, xmax = box[0], box[1], box[2], box[3]
    h = ymax - ymin
    w = xmax - xmin
    area = h * w
    tol = 0.2
    patch_size = jnp.floor(jnp.sqrt(area * scale))
    k1, k2 = jax.random.split(key)
    orig_y = ymin + h / 2.0 + jax.random.uniform(
        k1, (), minval=-tol * h / 2.0, maxval=tol * h / 2.0)
    orig_x = xmin + w / 2.0 + jax.random.uniform(
        k2, (), minval=-tol * w / 2.0, maxval=tol * w / 2.0)
    ymin_p = jnp.maximum(orig_y - patch_size / 2.0, 0.0)
    xmin_p = jnp.maximum(orig_x - patch_size / 2.0, 0.0)
    ymin_p = jnp.where(ymin_p + patch_size > img_h, img_h - patch_size, ymin_p)
    xmin_p = jnp.where(xmin_p + patch_size > img_w, img_w - patch_size, xmin_p)
    return jnp.stack([ymin_p, xmin_p, patch_size, patch_size])


def _rotl(x, d):
    return lax.shift_left(x, jnp.uint32(d)) | lax.shift_right_logical(
        x, jnp.uint32(32 - d))


def _threefry_bits(k0, k1, v):
    ks0 = k0
    ks1 = k1
    ks2 = jnp.uint32(0x1BD11BDA) ^ ks0 ^ ks1
    ks = (ks0, ks1, ks2)
    rots = ((13, 15, 26, 6), (17, 29, 16, 24))
    x0 = jnp.full_like(v, ks0)
    x1 = v + ks1
    for i in range(5):
        for r in rots[i % 2]:
            x0 = x0 + x1
            x1 = _rotl(x1, r) ^ x0
        x0 = x0 + ks[(i + 1) % 3]
        x1 = x1 + ks[(i + 2) % 3] + jnp.uint32(i + 1)
    return x0 ^ x1


def _patch_body(geo_ref, fs_ref, wb_ref, img_ref, rmat_ref, emat_ref,
                patch_ref, out_ref):
    b = pl.program_id(0)
    img = img_ref[0]
    mean_img = jnp.mean(img)

    p0 = jnp.clip(wb_ref[b, 0] * patch_ref[0] + wb_ref[b, 3], -1.0, 1.0)
    p1 = jnp.clip(wb_ref[b, 1] * patch_ref[1] + wb_ref[b, 4], -1.0, 1.0)
    p2 = jnp.clip(wb_ref[b, 2] * patch_ref[2] + wb_ref[b, 5], -1.0, 1.0)
    mean_p = (jnp.sum(p0) + jnp.sum(p1) + jnp.sum(p2)) / (3.0 * p0.size)
    delta = mean_img - mean_p

    out_ref[0] = img

    nb = geo_ref.shape[1]
    for ni in range(nb):
        wy = geo_ref[b, ni, 0]
        wx = geo_ref[b, ni, 1]
        oy = geo_ref[b, ni, 2]
        ox = geo_ref[b, ni, 3]
        s = geo_ref[b, ni, 4]
        sidx = geo_ref[b, ni, 5]
        live = geo_ref[b, ni, 6]
        k0 = geo_ref[b, ni, 7]
        k1 = geo_ref[b, ni, 8]
        shift = fs_ref[b, ni]

        @pl.when(live == 1)
        def _():
            rmat = rmat_ref[sidx]
            q0 = jnp.dot(rmat, p0, preferred_element_type=jnp.float32)
            q1 = jnp.dot(rmat, p1, preferred_element_type=jnp.float32)
            q2 = jnp.dot(rmat, p2, preferred_element_type=jnp.float32)
            tile = (jnp.dot(q0, emat_ref[0, sidx],
                            preferred_element_type=jnp.float32)
                    + jnp.dot(q1, emat_ref[1, sidx],
                              preferred_element_type=jnp.float32)
                    + jnp.dot(q2, emat_ref[2, sidx],
                              preferred_element_type=jnp.float32))

            ri = lax.broadcasted_iota(jnp.uint32, (TS, TW), 0)
            ci = lax.broadcasted_iota(jnp.uint32, (TS, TW), 1)
            v = jnp.uint32(3) * jnp.uint32(s) * ri + ci
            bits = _threefry_bits(jnp.uint32(k0), jnp.uint32(k1), v)
            f = lax.bitcast_convert_type(
                lax.shift_right_logical(bits, jnp.uint32(9))
                | jnp.uint32(0x3F800000), jnp.float32) - 1.0
            noise = jnp.maximum(jnp.float32(-0.01), f * 0.02 - 0.01)

            tile = jnp.clip(tile + (delta + shift) + noise, -1.0, 1.0)

            canvas = jnp.pad(tile, ((0, WR - TS), (0, WC - TW)))
            rolled = pltpu.roll(pltpu.roll(canvas, oy, 0), ox, 1)
            rows = lax.broadcasted_iota(jnp.int32, (WR, WC), 0)
            cols = lax.broadcasted_iota(jnp.int32, (WR, WC), 1)
            valid = ((rows >= oy) & (rows < oy + s)
                     & (cols >= ox) & (cols < ox + 3 * s))
            wya = pl.multiple_of(wy, 8)
            wxa = pl.multiple_of(wx, 128)
            win = out_ref[0, pl.ds(wya, WR), pl.ds(wxa, WC)]
            out_ref[0, pl.ds(wya, WR), pl.ds(wxa, WC)] = jnp.where(
                valid, rolled, win)


def kernel(boxes, images, patch, scale):
    B, H, W, C = images.shape
    NB = boxes.shape[1]
    PS = patch.shape[0]
    WF = W * C

    base = jax.random.key(42)
    kb = jax.vmap(lambda i: jax.random.fold_in(base, i))(jnp.arange(B))
    ks = jax.vmap(lambda k: jax.random.split(k, 4))(kb)
    kw, kbias, kboxes, knoise = ks[:, 0], ks[:, 1], ks[:, 2], ks[:, 3]

    w = jax.vmap(lambda k: jax.random.normal(k, (3,)))(kw) * 0.01 + 0.8
    bvec = jax.vmap(lambda k: jax.random.normal(k, (3,)))(kbias) * 0.01 - 0.2
    wb = jnp.concatenate([w, bvec], axis=1).astype(jnp.float32)

    def per_box(kboxes_b, knoise_b, boxes_b, ni):
        pb = _create_box(jax.random.fold_in(kboxes_b, ni),
                         float(H), float(W), boxes_b[ni], scale)
        ps, pw_f = pb[2], pb[3]
        ph = jnp.floor(ps).astype(jnp.int32)
        y0 = jnp.floor(pb[0]).astype(jnp.int32)
        x0 = jnp.floor(pb[1]).astype(jnp.int32)
        sidx = jnp.clip(ph - SMIN, 0, NSIZES - 1)
        s = sidx + SMIN
        live = jnp.logical_not(
            (ps * pw_f <= MIN_PATCH_AREA) | (ph <= 0)).astype(jnp.int32)
        kk1, kk2 = jax.random.split(jax.random.fold_in(knoise_b, ni))
        shift = jax.random.uniform(kk2, (), minval=-0.3, maxval=0.3)
        kd = jax.random.key_data(kk1).astype(jnp.uint32)
        wy = jnp.clip((y0 // 8) * 8, 0, H - WR)
        wx = jnp.clip(((3 * x0) // 128) * 128, 0, WF - WC)
        oy = jnp.clip(y0 - wy, 0, WR - 1)
        ox = jnp.clip(3 * x0 - wx, 0, WC - 3)
        geo = jnp.stack([wy, wx, oy, ox, s, sidx, live,
                         kd[0].astype(jnp.int32), kd[1].astype(jnp.int32)])
        return geo, shift

    geo, shift = jax.vmap(
        lambda kb_, kn_, bx_: jax.vmap(
            lambda ni: per_box(kb_, kn_, bx_, ni))(jnp.arange(NB))
    )(kboxes, knoise, boxes)
    geo = geo.astype(jnp.int32)
    shift = shift.astype(jnp.float32)

    rmats_np = _resize_mats(PS)
    rmats = jnp.asarray(rmats_np)
    emats = jnp.asarray(_interleave_mats(rmats_np))
    patch_pl = patch.transpose(2, 0, 1)
    img_flat = images.reshape(B, H, WF)

    out = pl.pallas_call(
        _patch_body,
        grid=(B,),
        in_specs=[
            pl.BlockSpec(memory_space=pltpu.SMEM),
            pl.BlockSpec(memory_space=pltpu.SMEM),
            pl.BlockSpec(memory_space=pltpu.SMEM),
            pl.BlockSpec((1, H, WF), lambda b: (b, 0, 0)),
            pl.BlockSpec((NSIZES, TS, PS), lambda b: (0, 0, 0)),
            pl.BlockSpec((C, NSIZES, PS, TW), lambda b: (0, 0, 0, 0)),
            pl.BlockSpec((C, PS, PS), lambda b: (0, 0, 0)),
        ],
        out_specs=pl.BlockSpec((1, H, WF), lambda b: (b, 0, 0)),
        out_shape=jax.ShapeDtypeStruct((B, H, WF), jnp.float32),
    )(geo, shift, wb, img_flat, rmats, emats, patch_pl)

    return out.reshape(B, H, W, C)


# speedup vs baseline: 451.2236x; 2.7277x over previous
import numpy as np
import jax
import jax.numpy as jnp
from jax import lax
from jax.experimental import pallas as pl
from jax.experimental.pallas import tpu as pltpu

MIN_PATCH_AREA = 60.0
SMIN, SMAX = 20, 72
NSIZES = SMAX - SMIN + 1
TS = 72
WR = 128
WC = 256


def _resize_mats(insz: int) -> np.ndarray:
    out = np.zeros((NSIZES, TS, insz), np.float32)
    for i, s in enumerate(range(SMIN, SMAX + 1)):
        scale = s / insz
        kernel_scale = min(scale, 1.0)
        sample_f = (np.arange(s) + 0.5) / scale - 0.5
        x = np.abs(sample_f[None, :] - np.arange(insz)[:, None]) * kernel_scale
        w = np.maximum(0.0, 1.0 - x)
        total = w.sum(axis=0, keepdims=True)
        w = np.where(np.abs(total) > 1000.0 * np.finfo(np.float32).eps,
                     w / np.where(total == 0, 1, total), 0.0)
        ok = (sample_f >= -0.5) & (sample_f <= insz - 0.5)
        w = np.where(ok[None, :], w, 0.0)
        out[i, :s, :] = w.T.astype(np.float32)
    return out


def _create_box(key, img_h, img_w, box, scale):
    ymin, xmin, ymax, xmax = box[0], box[1], box[2], box[3]
    h = ymax - ymin
    w = xmax - xmin
    area = h * w
    tol = 0.2
    patch_size = jnp.floor(jnp.sqrt(area * scale))
    k1, k2 = jax.random.split(key)
    orig_y = ymin + h / 2.0 + jax.random.uniform(
        k1, (), minval=-tol * h / 2.0, maxval=tol * h / 2.0)
    orig_x = xmin + w / 2.0 + jax.random.uniform(
        k2, (), minval=-tol * w / 2.0, maxval=tol * w / 2.0)
    ymin_p = jnp.maximum(orig_y - patch_size / 2.0, 0.0)
    xmin_p = jnp.maximum(orig_x - patch_size / 2.0, 0.0)
    ymin_p = jnp.where(ymin_p + patch_size > img_h, img_h - patch_size, ymin_p)
    xmin_p = jnp.where(xmin_p + patch_size > img_w, img_w - patch_size, xmin_p)
    return jnp.stack([ymin_p, xmin_p, patch_size, patch_size])


def _rotl(x, d):
    return lax.shift_left(x, jnp.uint32(d)) | lax.shift_right_logical(
        x, jnp.uint32(32 - d))


def _threefry_bits(k0, k1, v):
    ks0 = k0
    ks1 = k1
    ks2 = jnp.uint32(0x1BD11BDA) ^ ks0 ^ ks1
    ks = (ks0, ks1, ks2)
    rots = ((13, 15, 26, 6), (17, 29, 16, 24))
    x0 = jnp.full_like(v, ks0)
    x1 = v + ks1
    for i in range(5):
        for r in rots[i % 2]:
            x0 = x0 + x1
            x1 = _rotl(x1, r) ^ x0
        x0 = x0 + ks[(i + 1) % 3]
        x1 = x1 + ks[(i + 2) % 3] + jnp.uint32(i + 1)
    return x0 ^ x1


def _patch_body(geo_ref, fs_ref, wb_ref, img_ref, rmat_ref, rmt_ref,
                patch_ref, out_ref):
    b = pl.program_id(0)
    img = img_ref[0]
    mean_img = jnp.mean(img)

    p0 = jnp.clip(wb_ref[b, 0] * patch_ref[0] + wb_ref[b, 3], -1.0, 1.0)
    p1 = jnp.clip(wb_ref[b, 1] * patch_ref[1] + wb_ref[b, 4], -1.0, 1.0)
    p2 = jnp.clip(wb_ref[b, 2] * patch_ref[2] + wb_ref[b, 5], -1.0, 1.0)
    mean_p = (jnp.sum(p0) + jnp.sum(p1) + jnp.sum(p2)) / (3.0 * p0.size)
    delta = mean_img - mean_p

    out_ref[0] = img

    nb = geo_ref.shape[1]
    for ni in range(nb):
        wy = geo_ref[b, ni, 0]
        wx = geo_ref[b, ni, 1]
        oy = geo_ref[b, ni, 2]
        ox = geo_ref[b, ni, 3]
        s = geo_ref[b, ni, 4]
        sidx = geo_ref[b, ni, 5]
        live = geo_ref[b, ni, 6]
        k0 = geo_ref[b, ni, 7]
        k1 = geo_ref[b, ni, 8]
        shift = fs_ref[b, ni]

        @pl.when(live == 1)
        def _():
            rmat = rmat_ref[sidx]
            rmt = rmt_ref[sidx]
            tiles = [
                jnp.dot(jnp.dot(rmat, p, preferred_element_type=jnp.float32),
                        rmt, preferred_element_type=jnp.float32)
                for p in (p0, p1, p2)
            ]

            ri = lax.broadcasted_iota(jnp.uint32, (TS, 3 * TS), 0)
            ci = lax.broadcasted_iota(jnp.uint32, (TS, 3 * TS), 1)
            cch = ((ci >= TS).astype(jnp.uint32)
                   + (ci >= 2 * TS).astype(jnp.uint32))
            v = (jnp.uint32(3) * (jnp.uint32(s) * ri + ci - jnp.uint32(TS) * cch)
                 + cch)
            bits = _threefry_bits(jnp.uint32(k0), jnp.uint32(k1), v)
            f = lax.bitcast_convert_type(
                lax.shift_right_logical(bits, jnp.uint32(9))
                | jnp.uint32(0x3F800000), jnp.float32) - 1.0
            noise = jnp.maximum(jnp.float32(-0.01), f * 0.02 - 0.01)

            rows = lax.broadcasted_iota(jnp.int32, (WR, WC), 0)
            cols = lax.broadcasted_iota(jnp.int32, (WR, WC), 1)
            valid = ((rows >= oy) & (rows < oy + s)
                     & (cols >= ox) & (cols < ox + s))
            wya = pl.multiple_of(wy, 8)
            wxa = pl.multiple_of(wx, 128)
            for c in range(3):
                tile = jnp.clip(
                    tiles[c] + (delta + shift) + noise[:, c * TS:(c + 1) * TS],
                    -1.0, 1.0)
                canvas = jnp.pad(tile, ((0, WR - TS), (0, WC - TS)))
                rolled = pltpu.roll(pltpu.roll(canvas, oy, 0), ox, 1)
                win = out_ref[0, c, pl.ds(wya, WR), pl.ds(wxa, WC)]
                out_ref[0, c, pl.ds(wya, WR), pl.ds(wxa, WC)] = jnp.where(
                    valid, rolled, win)


def kernel(boxes, images, patch, scale):
    B, H, W, C = images.shape
    NB = boxes.shape[1]
    PS = patch.shape[0]

    base = jax.random.key(42)
    kb = jax.vmap(lambda i: jax.random.fold_in(base, i))(jnp.arange(B))
    ks = jax.vmap(lambda k: jax.random.split(k, 4))(kb)
    kw, kbias, kboxes, knoise = ks[:, 0], ks[:, 1], ks[:, 2], ks[:, 3]

    w = jax.vmap(lambda k: jax.random.normal(k, (3,)))(kw) * 0.01 + 0.8
    bvec = jax.vmap(lambda k: jax.random.normal(k, (3,)))(kbias) * 0.01 - 0.2
    wb = jnp.concatenate([w, bvec], axis=1).astype(jnp.float32)

    def per_box(kboxes_b, knoise_b, boxes_b, ni):
        pb = _create_box(jax.random.fold_in(kboxes_b, ni),
                         float(H), float(W), boxes_b[ni], scale)
        ps, pw_f = pb[2], pb[3]
        ph = jnp.floor(ps).astype(jnp.int32)
        y0 = jnp.floor(pb[0]).astype(jnp.int32)
        x0 = jnp.floor(pb[1]).astype(jnp.int32)
        sidx = jnp.clip(ph - SMIN, 0, NSIZES - 1)
        s = sidx + SMIN
        live = jnp.logical_not(
            (ps * pw_f <= MIN_PATCH_AREA) | (ph <= 0)).astype(jnp.int32)
        kk1, kk2 = jax.random.split(jax.random.fold_in(knoise_b, ni))
        shift = jax.random.uniform(kk2, (), minval=-0.3, maxval=0.3)
        kd = jax.random.key_data(kk1).astype(jnp.uint32)
        wy = jnp.clip((y0 // 8) * 8, 0, H - WR)
        wx = jnp.clip((x0 // 128) * 128, 0, W - WC)
        oy = jnp.clip(y0 - wy, 0, WR - 1)
        ox = jnp.clip(x0 - wx, 0, WC - 1)
        geo = jnp.stack([wy, wx, oy, ox, s, sidx, live,
                         kd[0].astype(jnp.int32), kd[1].astype(jnp.int32)])
        return geo, shift

    geo, shift = jax.vmap(
        lambda kb_, kn_, bx_: jax.vmap(
            lambda ni: per_box(kb_, kn_, bx_, ni))(jnp.arange(NB))
    )(kboxes, knoise, boxes)
    geo = geo.astype(jnp.int32)
    shift = shift.astype(jnp.float32)

    rmats_np = _resize_mats(PS)
    rmats = jnp.asarray(rmats_np)
    rmts = jnp.asarray(np.ascontiguousarray(
        rmats_np.transpose(0, 2, 1)))
    patch_pl = patch.transpose(2, 0, 1)
    img_pl = images.transpose(0, 3, 1, 2)

    out = pl.pallas_call(
        _patch_body,
        grid=(B,),
        in_specs=[
            pl.BlockSpec(memory_space=pltpu.SMEM),
            pl.BlockSpec(memory_space=pltpu.SMEM),
            pl.BlockSpec(memory_space=pltpu.SMEM),
            pl.BlockSpec((1, C, H, W), lambda b: (b, 0, 0, 0)),
            pl.BlockSpec((NSIZES, TS, PS), lambda b: (0, 0, 0)),
            pl.BlockSpec((NSIZES, PS, TS), lambda b: (0, 0, 0)),
            pl.BlockSpec((C, PS, PS), lambda b: (0, 0, 0)),
        ],
        out_specs=pl.BlockSpec((1, C, H, W), lambda b: (b, 0, 0, 0)),
        out_shape=jax.ShapeDtypeStruct((B, C, H, W), jnp.float32),
    )(geo, shift, wb, img_pl, rmats, rmts, patch_pl)

    return out.transpose(0, 2, 3, 1)


# speedup vs baseline: 453.3458x; 1.0047x over previous
import numpy as np
import jax
import jax.numpy as jnp
from jax import lax
from jax.experimental import pallas as pl
from jax.experimental.pallas import tpu as pltpu

MIN_PATCH_AREA = 60.0
SMIN, SMAX = 20, 72
NSIZES = SMAX - SMIN + 1
TS = 72
WR = 128
WC = 256


def _resize_mats(insz: int) -> np.ndarray:
    out = np.zeros((NSIZES, TS, insz), np.float32)
    for i, s in enumerate(range(SMIN, SMAX + 1)):
        scale = s / insz
        kernel_scale = min(scale, 1.0)
        sample_f = (np.arange(s) + 0.5) / scale - 0.5
        x = np.abs(sample_f[None, :] - np.arange(insz)[:, None]) * kernel_scale
        w = np.maximum(0.0, 1.0 - x)
        total = w.sum(axis=0, keepdims=True)
        w = np.where(np.abs(total) > 1000.0 * np.finfo(np.float32).eps,
                     w / np.where(total == 0, 1, total), 0.0)
        ok = (sample_f >= -0.5) & (sample_f <= insz - 0.5)
        w = np.where(ok[None, :], w, 0.0)
        out[i, :s, :] = w.T.astype(np.float32)
    return out


_U32 = np.uint32


def _np_rotl(x, d):
    return ((x << _U32(d)) | (x >> _U32(32 - d))) & _U32(0xFFFFFFFF)


def _np_threefry(k0, k1, x0, x1):
    ks0, ks1 = _U32(k0), _U32(k1)
    ks2 = _U32(0x1BD11BDA) ^ ks0 ^ ks1
    x0 = (np.asarray(x0, _U32) + ks0).astype(_U32)
    x1 = (np.asarray(x1, _U32) + ks1).astype(_U32)
    rots = ((13, 15, 26, 6), (17, 29, 16, 24))
    ks = (ks0, ks1, ks2)
    for i in range(5):
        for r in rots[i % 2]:
            x0 = (x0 + x1).astype(_U32)
            x1 = _np_rotl(x1, r) ^ x0
        x0 = (x0 + ks[(i + 1) % 3]).astype(_U32)
        x1 = (x1 + ks[(i + 2) % 3] + _U32(i + 1)).astype(_U32)
    return x0, x1


def _np_fold_in(key, d):
    o0, o1 = _np_threefry(key[0], key[1], _U32(0), _U32(d))
    return np.array([o0, o1], _U32)


def _np_split(key, n):
    b1, b2 = _np_threefry(key[0], key[1], np.zeros(n, _U32),
                          np.arange(n, dtype=_U32))
    return np.stack([b1, b2], axis=1)


def _np_u01(key, n=1):
    b1, b2 = _np_threefry(key[0], key[1], np.zeros(n, _U32),
                          np.arange(n, dtype=_U32))
    bits = b1 ^ b2
    return (((bits >> _U32(9)) | _U32(0x3F800000)).view(np.float32)
            - np.float32(1.0))


def _np_normal3(key):
    from statistics import NormalDist
    lo = np.nextafter(np.float32(-1.0), np.float32(0.0), dtype=np.float32)
    u = np.maximum(lo, (_np_u01(key, 3) * (np.float32(1.0) - lo)
                        + lo).astype(np.float32))
    nd = NormalDist()
    return np.array([nd.inv_cdf((float(x) + 1.0) / 2.0) for x in u],
                    np.float32)


def _np_draws(B, NB):
    base = np.array([0, 42], _U32)
    wb = np.zeros((B, 6), np.float32)
    u1 = np.zeros((B, NB), np.float32)
    u2 = np.zeros((B, NB), np.float32)
    shift = np.zeros((B, NB), np.float32)
    kd = np.zeros((B, NB, 2), _U32)
    for bi in range(B):
        kb = _np_fold_in(base, bi)
        kw, kbias, kboxes, knoise = _np_split(kb, 4)
        wb[bi, :3] = _np_normal3(kw) * 0.01 + 0.8
        wb[bi, 3:] = _np_normal3(kbias) * 0.01 - 0.2
        for ni in range(NB):
            kc = _np_fold_in(kboxes, ni)
            k1, k2 = _np_split(kc, 2)
            u1[bi, ni] = _np_u01(k1)[0]
            u2[bi, ni] = _np_u01(k2)[0]
            kn = _np_fold_in(knoise, ni)
            kk1, kk2 = _np_split(kn, 2)
            kd[bi, ni] = kk1
            shift[bi, ni] = max(np.float32(-0.3),
                                np.float32(_np_u01(kk2)[0] * 0.6 - 0.3))
    return wb, u1, u2, shift, kd


def _rotl(x, d):
    return lax.shift_left(x, jnp.uint32(d)) | lax.shift_right_logical(
        x, jnp.uint32(32 - d))


def _threefry_bits(k0, k1, v):
    ks0 = k0
    ks1 = k1
    ks2 = jnp.uint32(0x1BD11BDA) ^ ks0 ^ ks1
    ks = (ks0, ks1, ks2)
    rots = ((13, 15, 26, 6), (17, 29, 16, 24))
    x0 = jnp.full_like(v, ks0)
    x1 = v + ks1
    for i in range(5):
        for r in rots[i % 2]:
            x0 = x0 + x1
            x1 = _rotl(x1, r) ^ x0
        x0 = x0 + ks[(i + 1) % 3]
        x1 = x1 + ks[(i + 2) % 3] + jnp.uint32(i + 1)
    return x0 ^ x1


def _patch_body(geo_ref, fs_ref, wb_ref, img_ref, rmat_ref, rmt_ref,
                patch_ref, out_ref):
    b = pl.program_id(0)
    img = img_ref[0]
    mean_img = jnp.mean(img)

    p0 = jnp.clip(wb_ref[b, 0] * patch_ref[0] + wb_ref[b, 3], -1.0, 1.0)
    p1 = jnp.clip(wb_ref[b, 1] * patch_ref[1] + wb_ref[b, 4], -1.0, 1.0)
    p2 = jnp.clip(wb_ref[b, 2] * patch_ref[2] + wb_ref[b, 5], -1.0, 1.0)
    mean_p = (jnp.sum(p0) + jnp.sum(p1) + jnp.sum(p2)) / (3.0 * p0.size)
    delta = mean_img - mean_p

    out_ref[0] = img

    nb = geo_ref.shape[1]
    for ni in range(nb):
        wy = geo_ref[b, ni, 0]
        wx = geo_ref[b, ni, 1]
        oy = geo_ref[b, ni, 2]
        ox = geo_ref[b, ni, 3]
        s = geo_ref[b, ni, 4]
        sidx = geo_ref[b, ni, 5]
        live = geo_ref[b, ni, 6]
        k0 = geo_ref[b, ni, 7]
        k1 = geo_ref[b, ni, 8]
        shift = fs_ref[b, ni]

        @pl.when(live == 1)
        def _():
            rmat = rmat_ref[sidx]
            rmt = rmt_ref[sidx]
            tiles = [
                jnp.dot(jnp.dot(rmat, p, preferred_element_type=jnp.float32),
                        rmt, preferred_element_type=jnp.float32)
                for p in (p0, p1, p2)
            ]

            ri = lax.broadcasted_iota(jnp.uint32, (TS, 3 * TS), 0)
            ci = lax.broadcasted_iota(jnp.uint32, (TS, 3 * TS), 1)
            cch = ((ci >= TS).astype(jnp.uint32)
                   + (ci >= 2 * TS).astype(jnp.uint32))
            v = (jnp.uint32(3) * (jnp.uint32(s) * ri + ci - jnp.uint32(TS) * cch)
                 + cch)
            bits = _threefry_bits(jnp.uint32(k0), jnp.uint32(k1), v)
            f = lax.bitcast_convert_type(
                lax.shift_right_logical(bits, jnp.uint32(9))
                | jnp.uint32(0x3F800000), jnp.float32) - 1.0
            noise = jnp.maximum(jnp.float32(-0.01), f * 0.02 - 0.01)

            rows = lax.broadcasted_iota(jnp.int32, (WR, WC), 0)
            cols = lax.broadcasted_iota(jnp.int32, (WR, WC), 1)
            valid = ((rows >= oy) & (rows < oy + s)
                     & (cols >= ox) & (cols < ox + s))
            wya = pl.multiple_of(wy, 8)
            wxa = pl.multiple_of(wx, 128)
            for c in range(3):
                tile = jnp.clip(
                    tiles[c] + (delta + shift) + noise[:, c * TS:(c + 1) * TS],
                    -1.0, 1.0)
                canvas = jnp.pad(tile, ((0, WR - TS), (0, WC - TS)))
                rolled = pltpu.roll(pltpu.roll(canvas, oy, 0), ox, 1)
                win = out_ref[0, c, pl.ds(wya, WR), pl.ds(wxa, WC)]
                out_ref[0, c, pl.ds(wya, WR), pl.ds(wxa, WC)] = jnp.where(
                    valid, rolled, win)


def kernel(boxes, images, patch, scale):
    B, H, W, C = images.shape
    NB = boxes.shape[1]
    PS = patch.shape[0]

    wb_np, u1_np, u2_np, shift_np, kd_np = _np_draws(B, NB)
    wb = jnp.asarray(wb_np)
    shift = jnp.asarray(shift_np)
    u1 = jnp.asarray(u1_np)
    u2 = jnp.asarray(u2_np)

    tol = 0.2
    ymin, xmin = boxes[..., 0], boxes[..., 1]
    ymax, xmax = boxes[..., 2], boxes[..., 3]
    hh = ymax - ymin
    ww = xmax - xmin
    area = hh * ww
    ps = jnp.floor(jnp.sqrt(area * scale))
    min_y, max_y = -tol * hh / 2.0, tol * hh / 2.0
    min_x, max_x = -tol * ww / 2.0, tol * ww / 2.0
    orig_y = ymin + hh / 2.0 + jnp.maximum(min_y, u1 * (max_y - min_y) + min_y)
    orig_x = xmin + ww / 2.0 + jnp.maximum(min_x, u2 * (max_x - min_x) + min_x)
    ymin_p = jnp.maximum(orig_y - ps / 2.0, 0.0)
    xmin_p = jnp.maximum(orig_x - ps / 2.0, 0.0)
    ymin_p = jnp.where(ymin_p + ps > float(H), float(H) - ps, ymin_p)
    xmin_p = jnp.where(xmin_p + ps > float(W), float(W) - ps, xmin_p)
    ph = jnp.floor(ps).astype(jnp.int32)
    y0 = jnp.floor(ymin_p).astype(jnp.int32)
    x0 = jnp.floor(xmin_p).astype(jnp.int32)
    sidx = jnp.clip(ph - SMIN, 0, NSIZES - 1)
    s = sidx + SMIN
    live = jnp.logical_not(
        (ps * ps <= MIN_PATCH_AREA) | (ph <= 0)).astype(jnp.int32)
    wy = jnp.clip((y0 // 8) * 8, 0, H - WR)
    wx = jnp.clip((x0 // 128) * 128, 0, W - WC)
    oy = jnp.clip(y0 - wy, 0, WR - 1)
    ox = jnp.clip(x0 - wx, 0, WC - 1)
    geo = jnp.stack(
        [wy, wx, oy, ox, s, sidx, live,
         jnp.asarray(kd_np[..., 0].astype(np.int64).astype(np.int32)),
         jnp.asarray(kd_np[..., 1].astype(np.int64).astype(np.int32))],
        axis=-1).astype(jnp.int32)

    rmats_np = _resize_mats(PS)
    rmats = jnp.asarray(rmats_np)
    rmts = jnp.asarray(np.ascontiguousarray(
        rmats_np.transpose(0, 2, 1)))
    patch_pl = patch.transpose(2, 0, 1)
    img_pl = images.transpose(0, 3, 1, 2)

    out = pl.pallas_call(
        _patch_body,
        grid=(B,),
        in_specs=[
            pl.BlockSpec(memory_space=pltpu.SMEM),
            pl.BlockSpec(memory_space=pltpu.SMEM),
            pl.BlockSpec(memory_space=pltpu.SMEM),
            pl.BlockSpec((1, C, H, W), lambda b: (b, 0, 0, 0)),
            pl.BlockSpec((NSIZES, TS, PS), lambda b: (0, 0, 0)),
            pl.BlockSpec((NSIZES, PS, TS), lambda b: (0, 0, 0)),
            pl.BlockSpec((C, PS, PS), lambda b: (0, 0, 0)),
        ],
        out_specs=pl.BlockSpec((1, C, H, W), lambda b: (b, 0, 0, 0)),
        out_shape=jax.ShapeDtypeStruct((B, C, H, W), jnp.float32),
    )(geo, shift, wb, img_pl, rmats, rmts, patch_pl)

    return out.transpose(0, 2, 3, 1)


# speedup vs baseline: 536.7019x; 1.1839x over previous
import numpy as np
import jax
import jax.numpy as jnp
from jax import lax
from jax.experimental import pallas as pl
from jax.experimental.pallas import tpu as pltpu

MIN_PATCH_AREA = 60.0
SMIN, SMAX = 20, 72
NSIZES = SMAX - SMIN + 1
TS = 72


def _resize_mats(insz: int) -> np.ndarray:
    out = np.zeros((NSIZES, TS, insz), np.float32)
    for i, s in enumerate(range(SMIN, SMAX + 1)):
        scale = s / insz
        kernel_scale = min(scale, 1.0)
        sample_f = (np.arange(s) + 0.5) / scale - 0.5
        x = np.abs(sample_f[None, :] - np.arange(insz)[:, None]) * kernel_scale
        w = np.maximum(0.0, 1.0 - x)
        total = w.sum(axis=0, keepdims=True)
        w = np.where(np.abs(total) > 1000.0 * np.finfo(np.float32).eps,
                     w / np.where(total == 0, 1, total), 0.0)
        ok = (sample_f >= -0.5) & (sample_f <= insz - 0.5)
        w = np.where(ok[None, :], w, 0.0)
        out[i, :s, :] = w.T.astype(np.float32)
    return out


_U32 = np.uint32


def _np_rotl(x, d):
    return ((x << _U32(d)) | (x >> _U32(32 - d))) & _U32(0xFFFFFFFF)


def _np_threefry(k0, k1, x0, x1):
    ks0, ks1 = _U32(k0), _U32(k1)
    ks2 = _U32(0x1BD11BDA) ^ ks0 ^ ks1
    x0 = (np.asarray(x0, _U32) + ks0).astype(_U32)
    x1 = (np.asarray(x1, _U32) + ks1).astype(_U32)
    rots = ((13, 15, 26, 6), (17, 29, 16, 24))
    ks = (ks0, ks1, ks2)
    for i in range(5):
        for r in rots[i % 2]:
            x0 = (x0 + x1).astype(_U32)
            x1 = _np_rotl(x1, r) ^ x0
        x0 = (x0 + ks[(i + 1) % 3]).astype(_U32)
        x1 = (x1 + ks[(i + 2) % 3] + _U32(i + 1)).astype(_U32)
    return x0, x1


def _np_fold_in(key, d):
    o0, o1 = _np_threefry(key[0], key[1], _U32(0), _U32(d))
    return np.array([o0, o1], _U32)


def _np_split(key, n):
    b1, b2 = _np_threefry(key[0], key[1], np.zeros(n, _U32),
                          np.arange(n, dtype=_U32))
    return np.stack([b1, b2], axis=1)


def _np_u01(key, n=1):
    b1, b2 = _np_threefry(key[0], key[1], np.zeros(n, _U32),
                          np.arange(n, dtype=_U32))
    bits = b1 ^ b2
    return (((bits >> _U32(9)) | _U32(0x3F800000)).view(np.float32)
            - np.float32(1.0))


def _np_normal3(key):
    from statistics import NormalDist
    lo = np.nextafter(np.float32(-1.0), np.float32(0.0), dtype=np.float32)
    u = np.maximum(lo, (_np_u01(key, 3) * (np.float32(1.0) - lo)
                        + lo).astype(np.float32))
    nd = NormalDist()
    return np.array([nd.inv_cdf((float(x) + 1.0) / 2.0) for x in u],
                    np.float32)


def _np_draws(B, NB):
    base = np.array([0, 42], _U32)
    wb = np.zeros((B, 6), np.float32)
    u1 = np.zeros((B, NB), np.float32)
    u2 = np.zeros((B, NB), np.float32)
    shift = np.zeros((B, NB), np.float32)
    kd = np.zeros((B, NB, 2), _U32)
    for bi in range(B):
        kb = _np_fold_in(base, bi)
        kw, kbias, kboxes, knoise = _np_split(kb, 4)
        wb[bi, :3] = _np_normal3(kw) * 0.01 + 0.8
        wb[bi, 3:] = _np_normal3(kbias) * 0.01 - 0.2
        for ni in range(NB):
            kc = _np_fold_in(kboxes, ni)
            k1, k2 = _np_split(kc, 2)
            u1[bi, ni] = _np_u01(k1)[0]
            u2[bi, ni] = _np_u01(k2)[0]
            kn = _np_fold_in(knoise, ni)
            kk1, kk2 = _np_split(kn, 2)
            kd[bi, ni] = kk1
            shift[bi, ni] = max(np.float32(-0.3),
                                np.float32(_np_u01(kk2)[0] * 0.6 - 0.3))
    return wb, u1, u2, shift, kd


def _rotl(x, d):
    return lax.shift_left(x, jnp.uint32(d)) | lax.shift_right_logical(
        x, jnp.uint32(32 - d))


def _threefry_bits(k0, k1, v):
    ks0 = k0
    ks1 = k1
    ks2 = jnp.uint32(0x1BD11BDA) ^ ks0 ^ ks1
    ks = (ks0, ks1, ks2)
    rots = ((13, 15, 26, 6), (17, 29, 16, 24))
    x0 = jnp.full_like(v, ks0)
    x1 = v + ks1
    for i in range(5):
        for r in rots[i % 2]:
            x0 = x0 + x1
            x1 = _rotl(x1, r) ^ x0
        x0 = x0 + ks[(i + 1) % 3]
        x1 = x1 + ks[(i + 2) % 3] + jnp.uint32(i + 1)
    return x0 ^ x1


def _patch_body(geo_ref, fs_ref, wb_ref, img_ref, rmat_ref, rmt_ref,
                patch_ref, out_ref):
    b = pl.program_id(0)
    H, W = img_ref.shape[2], img_ref.shape[3]
    img = img_ref[0]
    mean_img = jnp.mean(img)

    p0 = jnp.clip(wb_ref[b, 0] * patch_ref[0] + wb_ref[b, 3], -1.0, 1.0)
    p1 = jnp.clip(wb_ref[b, 1] * patch_ref[1] + wb_ref[b, 4], -1.0, 1.0)
    p2 = jnp.clip(wb_ref[b, 2] * patch_ref[2] + wb_ref[b, 5], -1.0, 1.0)
    mean_p = (jnp.sum(p0) + jnp.sum(p1) + jnp.sum(p2)) / (3.0 * p0.size)
    delta = mean_img - mean_p

    out_ref[0] = img

    nb = geo_ref.shape[1]
    for ni in range(nb):
        y0 = geo_ref[b, ni, 0]
        x0 = geo_ref[b, ni, 1]
        s = geo_ref[b, ni, 2]
        sidx = geo_ref[b, ni, 3]
        live = geo_ref[b, ni, 4]
        k0 = geo_ref[b, ni, 5]
        k1 = geo_ref[b, ni, 6]
        shift = fs_ref[b, ni]

        def do_box(SB, WRb, WCb):
            wy = jnp.minimum((y0 // 8) * 8, H - WRb)
            wx = jnp.minimum((x0 // 128) * 128, W - WCb)
            oy = y0 - wy
            ox = x0 - wx
            rmat = rmat_ref[sidx, :SB]
            rmt = rmt_ref[sidx, :, :SB]
            tiles = [
                jnp.dot(jnp.dot(rmat, p, preferred_element_type=jnp.float32),
                        rmt, preferred_element_type=jnp.float32)
                for p in (p0, p1, p2)
            ]

            ri = lax.broadcasted_iota(jnp.uint32, (SB, 3 * SB), 0)
            ci = lax.broadcasted_iota(jnp.uint32, (SB, 3 * SB), 1)
            cch = ((ci >= SB).astype(jnp.uint32)
                   + (ci >= 2 * SB).astype(jnp.uint32))
            v = (jnp.uint32(3) * (jnp.uint32(s) * ri + ci - jnp.uint32(SB) * cch)
                 + cch)
            bits = _threefry_bits(jnp.uint32(k0), jnp.uint32(k1), v)
            f = lax.bitcast_convert_type(
                lax.shift_right_logical(bits, jnp.uint32(9))
                | jnp.uint32(0x3F800000), jnp.float32) - 1.0
            noise = jnp.maximum(jnp.float32(-0.01), f * 0.02 - 0.01)

            rows = lax.broadcasted_iota(jnp.int32, (WRb, WCb), 0)
            cols = lax.broadcasted_iota(jnp.int32, (WRb, WCb), 1)
            valid = ((rows >= oy) & (rows < oy + s)
                     & (cols >= ox) & (cols < ox + s))
            wya = pl.multiple_of(wy, 8)
            wxa = pl.multiple_of(wx, 128)
            for c in range(3):
                tile = jnp.clip(
                    tiles[c] + (delta + shift) + noise[:, c * SB:(c + 1) * SB],
                    -1.0, 1.0)
                canvas = jnp.pad(tile, ((0, WRb - SB), (0, WCb - SB)))
                rolled = pltpu.roll(pltpu.roll(canvas, oy, 0), ox, 1)
                win = out_ref[0, c, pl.ds(wya, WRb), pl.ds(wxa, WCb)]
                out_ref[0, c, pl.ds(wya, WRb), pl.ds(wxa, WCb)] = jnp.where(
                    valid, rolled, win)

        @pl.when((live == 1) & (s <= 48))
        def _():
            do_box(48, 56, 256)

        @pl.when((live == 1) & (s > 48))
        def _():
            do_box(TS, 80, 256)


def kernel(boxes, images, patch, scale):
    B, H, W, C = images.shape
    NB = boxes.shape[1]
    PS = patch.shape[0]

    wb_np, u1_np, u2_np, shift_np, kd_np = _np_draws(B, NB)
    wb = jnp.asarray(wb_np)
    shift = jnp.asarray(shift_np)
    u1 = jnp.asarray(u1_np)
    u2 = jnp.asarray(u2_np)

    tol = 0.2
    ymin, xmin = boxes[..., 0], boxes[..., 1]
    ymax, xmax = boxes[..., 2], boxes[..., 3]
    hh = ymax - ymin
    ww = xmax - xmin
    area = hh * ww
    ps = jnp.floor(jnp.sqrt(area * scale))
    min_y, max_y = -tol * hh / 2.0, tol * hh / 2.0
    min_x, max_x = -tol * ww / 2.0, tol * ww / 2.0
    orig_y = ymin + hh / 2.0 + jnp.maximum(min_y, u1 * (max_y - min_y) + min_y)
    orig_x = xmin + ww / 2.0 + jnp.maximum(min_x, u2 * (max_x - min_x) + min_x)
    ymin_p = jnp.maximum(orig_y - ps / 2.0, 0.0)
    xmin_p = jnp.maximum(orig_x - ps / 2.0, 0.0)
    ymin_p = jnp.where(ymin_p + ps > float(H), float(H) - ps, ymin_p)
    xmin_p = jnp.where(xmin_p + ps > float(W), float(W) - ps, xmin_p)
    ph = jnp.floor(ps).astype(jnp.int32)
    y0 = jnp.floor(ymin_p).astype(jnp.int32)
    x0 = jnp.floor(xmin_p).astype(jnp.int32)
    sidx = jnp.clip(ph - SMIN, 0, NSIZES - 1)
    s = sidx + SMIN
    live = jnp.logical_not(
        (ps * ps <= MIN_PATCH_AREA) | (ph <= 0)).astype(jnp.int32)
    y0 = jnp.clip(y0, 0, H - s)
    x0 = jnp.clip(x0, 0, W - s)
    geo = jnp.stack(
        [y0, x0, s, sidx, live,
         jnp.asarray(kd_np[..., 0].astype(np.int64).astype(np.int32)),
         jnp.asarray(kd_np[..., 1].astype(np.int64).astype(np.int32))],
        axis=-1).astype(jnp.int32)

    rmats_np = _resize_mats(PS)
    rmats = jnp.asarray(rmats_np)
    rmts = jnp.asarray(np.ascontiguousarray(
        rmats_np.transpose(0, 2, 1)))
    patch_pl = patch.transpose(2, 0, 1)
    img_pl = images.transpose(0, 3, 1, 2)

    out = pl.pallas_call(
        _patch_body,
        grid=(B,),
        in_specs=[
            pl.BlockSpec(memory_space=pltpu.SMEM),
            pl.BlockSpec(memory_space=pltpu.SMEM),
            pl.BlockSpec(memory_space=pltpu.SMEM),
            pl.BlockSpec((1, C, H, W), lambda b: (b, 0, 0, 0)),
            pl.BlockSpec((NSIZES, TS, PS), lambda b: (0, 0, 0)),
            pl.BlockSpec((NSIZES, PS, TS), lambda b: (0, 0, 0)),
            pl.BlockSpec((C, PS, PS), lambda b: (0, 0, 0)),
        ],
        out_specs=pl.BlockSpec((1, C, H, W), lambda b: (b, 0, 0, 0)),
        out_shape=jax.ShapeDtypeStruct((B, C, H, W), jnp.float32),
    )(geo, shift, wb, img_pl, rmats, rmts, patch_pl)

    return out.transpose(0, 2, 3, 1)


# speedup vs baseline: 545.0138x; 1.0155x over previous
import numpy as np
import jax
import jax.numpy as jnp
from jax import lax
from jax.experimental import pallas as pl
from jax.experimental.pallas import tpu as pltpu

MIN_PATCH_AREA = 60.0
SMIN, SMAX = 20, 72
NSIZES = SMAX - SMIN + 1
TS = 72


def _resize_mats(insz: int) -> np.ndarray:
    out = np.zeros((NSIZES, TS, insz), np.float32)
    for i, s in enumerate(range(SMIN, SMAX + 1)):
        scale = s / insz
        kernel_scale = min(scale, 1.0)
        sample_f = (np.arange(s) + 0.5) / scale - 0.5
        x = np.abs(sample_f[None, :] - np.arange(insz)[:, None]) * kernel_scale
        w = np.maximum(0.0, 1.0 - x)
        total = w.sum(axis=0, keepdims=True)
        w = np.where(np.abs(total) > 1000.0 * np.finfo(np.float32).eps,
                     w / np.where(total == 0, 1, total), 0.0)
        ok = (sample_f >= -0.5) & (sample_f <= insz - 0.5)
        w = np.where(ok[None, :], w, 0.0)
        out[i, :s, :] = w.T.astype(np.float32)
    return out


_U32 = np.uint32


def _np_rotl(x, d):
    return ((x << _U32(d)) | (x >> _U32(32 - d))) & _U32(0xFFFFFFFF)


def _np_threefry(k0, k1, x0, x1):
    ks0, ks1 = _U32(k0), _U32(k1)
    ks2 = _U32(0x1BD11BDA) ^ ks0 ^ ks1
    x0 = (np.asarray(x0, _U32) + ks0).astype(_U32)
    x1 = (np.asarray(x1, _U32) + ks1).astype(_U32)
    rots = ((13, 15, 26, 6), (17, 29, 16, 24))
    ks = (ks0, ks1, ks2)
    for i in range(5):
        for r in rots[i % 2]:
            x0 = (x0 + x1).astype(_U32)
            x1 = _np_rotl(x1, r) ^ x0
        x0 = (x0 + ks[(i + 1) % 3]).astype(_U32)
        x1 = (x1 + ks[(i + 2) % 3] + _U32(i + 1)).astype(_U32)
    return x0, x1


def _np_fold_in(key, d):
    o0, o1 = _np_threefry(key[0], key[1], _U32(0), _U32(d))
    return np.array([o0, o1], _U32)


def _np_split(key, n):
    b1, b2 = _np_threefry(key[0], key[1], np.zeros(n, _U32),
                          np.arange(n, dtype=_U32))
    return np.stack([b1, b2], axis=1)


def _np_u01(key, n=1):
    b1, b2 = _np_threefry(key[0], key[1], np.zeros(n, _U32),
                          np.arange(n, dtype=_U32))
    bits = b1 ^ b2
    return (((bits >> _U32(9)) | _U32(0x3F800000)).view(np.float32)
            - np.float32(1.0))


def _np_normal3(key):
    from statistics import NormalDist
    lo = np.nextafter(np.float32(-1.0), np.float32(0.0), dtype=np.float32)
    u = np.maximum(lo, (_np_u01(key, 3) * (np.float32(1.0) - lo)
                        + lo).astype(np.float32))
    nd = NormalDist()
    return np.array([nd.inv_cdf((float(x) + 1.0) / 2.0) for x in u],
                    np.float32)


def _np_draws(B, NB):
    base = np.array([0, 42], _U32)
    wb = np.zeros((B, 6), np.float32)
    u1 = np.zeros((B, NB), np.float32)
    u2 = np.zeros((B, NB), np.float32)
    shift = np.zeros((B, NB), np.float32)
    kd = np.zeros((B, NB, 2), _U32)
    for bi in range(B):
        kb = _np_fold_in(base, bi)
        kw, kbias, kboxes, knoise = _np_split(kb, 4)
        wb[bi, :3] = _np_normal3(kw) * 0.01 + 0.8
        wb[bi, 3:] = _np_normal3(kbias) * 0.01 - 0.2
        for ni in range(NB):
            kc = _np_fold_in(kboxes, ni)
            k1, k2 = _np_split(kc, 2)
            u1[bi, ni] = _np_u01(k1)[0]
            u2[bi, ni] = _np_u01(k2)[0]
            kn = _np_fold_in(knoise, ni)
            kk1, kk2 = _np_split(kn, 2)
            kd[bi, ni] = kk1
            shift[bi, ni] = max(np.float32(-0.3),
                                np.float32(_np_u01(kk2)[0] * 0.6 - 0.3))
    return wb, u1, u2, shift, kd


def _rotl(x, d):
    return lax.shift_left(x, jnp.uint32(d)) | lax.shift_right_logical(
        x, jnp.uint32(32 - d))


def _threefry_bits(k0, k1, v):
    ks0 = k0
    ks1 = k1
    ks2 = jnp.uint32(0x1BD11BDA) ^ ks0 ^ ks1
    ks = (ks0, ks1, ks2)
    rots = ((13, 15, 26, 6), (17, 29, 16, 24))
    x0 = jnp.full_like(v, ks0)
    x1 = v + ks1
    for i in range(5):
        for r in rots[i % 2]:
            x0 = x0 + x1
            x1 = _rotl(x1, r) ^ x0
        x0 = x0 + ks[(i + 1) % 3]
        x1 = x1 + ks[(i + 2) % 3] + jnp.uint32(i + 1)
    return x0 ^ x1


def _patch_body(geo_ref, fs_ref, wb_ref, img_ref, rmat_ref, rmt_ref,
                patch_ref, cm48_ref, cm72_ref, out_ref, copy_sem):
    b = pl.program_id(0)
    H, W = img_ref.shape[2], img_ref.shape[3]
    cp = pltpu.make_async_copy(img_ref, out_ref, copy_sem)
    cp.start()
    img = img_ref[0]
    mean_img = jnp.mean(img)

    p0 = jnp.clip(wb_ref[b, 0] * patch_ref[0] + wb_ref[b, 3], -1.0, 1.0)
    p1 = jnp.clip(wb_ref[b, 1] * patch_ref[1] + wb_ref[b, 4], -1.0, 1.0)
    p2 = jnp.clip(wb_ref[b, 2] * patch_ref[2] + wb_ref[b, 5], -1.0, 1.0)
    mean_p = (jnp.sum(p0) + jnp.sum(p1) + jnp.sum(p2)) / (3.0 * p0.size)
    delta = mean_img - mean_p

    cp.wait()

    nb = geo_ref.shape[1]
    for ni in range(nb):
        y0 = geo_ref[b, ni, 0]
        x0 = geo_ref[b, ni, 1]
        s = geo_ref[b, ni, 2]
        sidx = geo_ref[b, ni, 3]
        live = geo_ref[b, ni, 4]
        k0 = geo_ref[b, ni, 5]
        k1 = geo_ref[b, ni, 6]
        shift = fs_ref[b, ni]

        def do_box(SB, WRb, WCb):
            wy = jnp.minimum((y0 // 8) * 8, H - WRb)
            wx = jnp.minimum((x0 // 128) * 128, W - WCb)
            oy = y0 - wy
            ox = x0 - wx
            rmat = rmat_ref[sidx, :SB]
            rmt = rmt_ref[sidx, :, :SB]
            tiles = [
                jnp.dot(jnp.dot(rmat, p, preferred_element_type=jnp.float32),
                        rmt, preferred_element_type=jnp.float32)
                for p in (p0, p1, p2)
            ]

            cm = (cm48_ref if SB == 48 else cm72_ref)[:, :]
            ri = lax.broadcasted_iota(jnp.uint32, (SB, 3 * SB), 0)
            v = jnp.uint32(3) * jnp.uint32(s) * ri + cm
            bits = _threefry_bits(jnp.uint32(k0), jnp.uint32(k1), v)
            f = lax.bitcast_convert_type(
                lax.shift_right_logical(bits, jnp.uint32(9))
                | jnp.uint32(0x3F800000), jnp.float32) - 1.0
            noise = jnp.maximum(jnp.float32(-0.01), f * 0.02 - 0.01)

            rows = lax.broadcasted_iota(jnp.int32, (WRb, WCb), 0)
            cols = lax.broadcasted_iota(jnp.int32, (WRb, WCb), 1)
            valid = ((rows >= oy) & (rows < oy + s)
                     & (cols >= ox) & (cols < ox + s))
            wya = pl.multiple_of(wy, 8)
            wxa = pl.multiple_of(wx, 128)
            for c in range(3):
                tile = jnp.clip(
                    tiles[c] + (delta + shift) + noise[:, c * SB:(c + 1) * SB],
                    -1.0, 1.0)
                canvas = jnp.pad(tile, ((0, WRb - SB), (0, WCb - SB)))
                rolled = pltpu.roll(pltpu.roll(canvas, oy, 0), ox, 1)
                win = out_ref[0, c, pl.ds(wya, WRb), pl.ds(wxa, WCb)]
                out_ref[0, c, pl.ds(wya, WRb), pl.ds(wxa, WCb)] = jnp.where(
                    valid, rolled, win)

        @pl.when((live == 1) & (s <= 48))
        def _():
            do_box(48, 56, 256)

        @pl.when((live == 1) & (s > 48))
        def _():
            do_box(TS, 80, 256)


def kernel(boxes, images, patch, scale):
    B, H, W, C = images.shape
    NB = boxes.shape[1]
    PS = patch.shape[0]

    wb_np, u1_np, u2_np, shift_np, kd_np = _np_draws(B, NB)
    wb = jnp.asarray(wb_np)
    shift = jnp.asarray(shift_np)
    u1 = jnp.asarray(u1_np)
    u2 = jnp.asarray(u2_np)

    tol = 0.2
    ymin, xmin = boxes[..., 0], boxes[..., 1]
    ymax, xmax = boxes[..., 2], boxes[..., 3]
    hh = ymax - ymin
    ww = xmax - xmin
    area = hh * ww
    ps = jnp.floor(jnp.sqrt(area * scale))
    min_y, max_y = -tol * hh / 2.0, tol * hh / 2.0
    min_x, max_x = -tol * ww / 2.0, tol * ww / 2.0
    orig_y = ymin + hh / 2.0 + jnp.maximum(min_y, u1 * (max_y - min_y) + min_y)
    orig_x = xmin + ww / 2.0 + jnp.maximum(min_x, u2 * (max_x - min_x) + min_x)
    ymin_p = jnp.maximum(orig_y - ps / 2.0, 0.0)
    xmin_p = jnp.maximum(orig_x - ps / 2.0, 0.0)
    ymin_p = jnp.where(ymin_p + ps > float(H), float(H) - ps, ymin_p)
    xmin_p = jnp.where(xmin_p + ps > float(W), float(W) - ps, xmin_p)
    ph = jnp.floor(ps).astype(jnp.int32)
    y0 = jnp.floor(ymin_p).astype(jnp.int32)
    x0 = jnp.floor(xmin_p).astype(jnp.int32)
    sidx = jnp.clip(ph - SMIN, 0, NSIZES - 1)
    s = sidx + SMIN
    live = jnp.logical_not(
        (ps * ps <= MIN_PATCH_AREA) | (ph <= 0)).astype(jnp.int32)
    y0 = jnp.clip(y0, 0, H - s)
    x0 = jnp.clip(x0, 0, W - s)
    geo = jnp.stack(
        [y0, x0, s, sidx, live,
         jnp.asarray(kd_np[..., 0].astype(np.int64).astype(np.int32)),
         jnp.asarray(kd_np[..., 1].astype(np.int64).astype(np.int32))],
        axis=-1).astype(jnp.int32)

    def colmap(SB):
        j = np.arange(3 * SB, dtype=np.uint32)
        c = (j >= SB).astype(_U32) + (j >= 2 * SB).astype(_U32)
        return (3 * (j - _U32(SB) * c) + c)[None, :].astype(_U32)

    rmats_np = _resize_mats(PS)
    rmats = jnp.asarray(rmats_np)
    rmts = jnp.asarray(np.ascontiguousarray(
        rmats_np.transpose(0, 2, 1)))
    patch_pl = patch.transpose(2, 0, 1)
    img_pl = images.transpose(0, 3, 1, 2)

    out = pl.pallas_call(
        _patch_body,
        grid=(B,),
        in_specs=[
            pl.BlockSpec(memory_space=pltpu.SMEM),
            pl.BlockSpec(memory_space=pltpu.SMEM),
            pl.BlockSpec(memory_space=pltpu.SMEM),
            pl.BlockSpec((1, C, H, W), lambda b: (b, 0, 0, 0)),
            pl.BlockSpec((NSIZES, TS, PS), lambda b: (0, 0, 0)),
            pl.BlockSpec((NSIZES, PS, TS), lambda b: (0, 0, 0)),
            pl.BlockSpec((C, PS, PS), lambda b: (0, 0, 0)),
            pl.BlockSpec((1, 144), lambda b: (0, 0)),
            pl.BlockSpec((1, 216), lambda b: (0, 0)),
        ],
        out_specs=pl.BlockSpec((1, C, H, W), lambda b: (b, 0, 0, 0)),
        out_shape=jax.ShapeDtypeStruct((B, C, H, W), jnp.float32),
        scratch_shapes=[pltpu.SemaphoreType.DMA],
    )(geo, shift, wb, img_pl, rmats, rmts, patch_pl,
      jnp.asarray(colmap(48)), jnp.asarray(colmap(72)))

    return out.transpose(0, 2, 3, 1)
